# baseline (speedup 1.0000x reference)
import functools

import jax
import jax.numpy as jnp
from jax import lax
from jax.experimental import pallas as pl
from jax.experimental.pallas import tpu as pltpu
from jax.experimental.pallas import tpu_sc as plsc

_FIELD_DIM = 100000
_IDX_ROW = 128
_BLK = 2048
_SUB = _BLK // 4

_NC = 2
_NS = 16
_NW = _NC * _NS


def _detile(tt):
    emb, n_rows = tt.shape
    n_blocks = pl.cdiv(n_rows, _BLK)

    def body(a0, a1, a2, a3, o_ref):
        o_ref[...] = jnp.concatenate(
            [a0[...].T, a1[...].T, a2[...].T, a3[...].T], axis=1)

    in_specs = [
        pl.BlockSpec((emb, _SUB), lambda i, a=a: (0, 4 * i + a))
        for a in range(4)
    ]
    return pl.pallas_call(
        body,
        grid=(n_blocks,),
        in_specs=in_specs,
        out_specs=pl.BlockSpec((_SUB, 128), lambda i: (i, 0)),
        out_shape=jax.ShapeDtypeStruct((n_blocks * _SUB, 128), jnp.float32),
    )(tt, tt, tt, tt)


def _sc_gather(x128, linv, nf):
    batch = x128.shape[0]
    emb = 32
    rows_w = batch // _NW
    n_g = rows_w // _IDX_ROW

    mesh = plsc.VectorSubcoreMesh(core_axis_name="c", subcore_axis_name="s")

    @functools.partial(
        pl.kernel,
        mesh=mesh,
        out_type=jax.ShapeDtypeStruct((batch, nf, emb), jnp.float32),
        scratch_types=[
            pltpu.VMEM((rows_w, 128), jnp.int32),
            pltpu.VMEM((rows_w,), jnp.int32),
            pltpu.VMEM((rows_w, 1, emb), jnp.float32),
            pltpu.SemaphoreType.DMA,
        ],
        compiler_params=pltpu.CompilerParams(
            use_tc_tiling_on_sc=False, needs_layout_passes=False),
    )
    def body(x_hbm, lin_hbm, out_hbm, xs_v, idx_v, rows_v, sem):
        wid = lax.axis_index("s") * _NC + lax.axis_index("c")
        b0 = wid * rows_w
        pltpu.sync_copy(x_hbm.at[pl.ds(b0, rows_w)], xs_v)

        def field_body(j, carry):
            jv = jnp.full((16,), 0, jnp.int32) + j
            off = j * _FIELD_DIM
            for k in range(rows_w // 16):
                riv = jax.lax.iota(jnp.int32, 16) + (k * 16)
                r = plsc.load_gather(xs_v, [riv, jv]) + off
                q = (
                    lax.shift_left(lax.shift_right_logical(r, 11), 11)
                    + lax.shift_left(r & jnp.int32(511), 2)
                    + (lax.shift_right_logical(r, 9) & jnp.int32(3))
                )
                idx_v[pl.ds(k * 16, 16)] = q
            copies = [
                pltpu.async_copy(
                    lin_hbm.at[idx_v.at[pl.ds(k * _IDX_ROW, _IDX_ROW)]],
                    rows_v.at[pl.ds(k * _IDX_ROW, _IDX_ROW), 0],
                    sem,
                )
                for k in range(n_g)
            ]
            for cp in copies:
                cp.wait()
            pltpu.sync_copy(rows_v, out_hbm.at[pl.ds(b0, rows_w), pl.ds(j, 1)])
            return carry

        lax.fori_loop(0, nf, field_body, 0)

    return body(x128, linv)


def kernel(x, table):
    nf = x.shape[1]
    x128 = jnp.pad(x, ((0, 0), (0, 128 - nf)))
    tt = table.T
    pad_cols = (-tt.shape[1]) % _BLK
    tt = jnp.pad(tt, ((0, 0), (0, pad_cols)))
    lin = _detile(tt)
    linv = lin.reshape(lin.shape[0] * 4, 32)
    return _sc_gather(x128, linv, nf)
